# baseline (speedup 1.0000x reference)
import functools

import jax
import jax.numpy as jnp
from jax import lax
from jax.experimental import pallas as pl
from jax.experimental.pallas import tpu as pltpu
from jax.experimental.pallas import tpu_sc as plsc

EMBED_DIM = 32
NUM_CORES = 2
NUM_SUBCORES = 16
NUM_WORKERS = NUM_CORES * NUM_SUBCORES

CHUNK = 128
NBUF = 4


@functools.lru_cache(maxsize=None)
def _make_kernel(total_rows):
    rows_per_w = total_rows // NUM_WORKERS
    nchunk = rows_per_w // CHUNK
    ngroup = nchunk // NBUF
    mesh = plsc.VectorSubcoreMesh(core_axis_name="c", subcore_axis_name="s")

    @functools.partial(
        pl.kernel,
        mesh=mesh,
        out_type=jax.ShapeDtypeStruct((total_rows, EMBED_DIM), jnp.float32),
        compiler_params=pltpu.CompilerParams(use_tc_tiling_on_sc=False),
        scratch_types=[
            pltpu.VMEM((nchunk, CHUNK), jnp.int32),
            pltpu.VMEM((NBUF, CHUNK, EMBED_DIM), jnp.float32),
            pltpu.SemaphoreType.DMA((NBUF,)),
        ],
    )
    def gather_kernel(idx_hbm, table_hbm, out_hbm, idx_v, rows_v, gsem):
        wid = lax.axis_index("s") * NUM_CORES + lax.axis_index("c")
        base = wid * rows_per_w
        pltpu.sync_copy(idx_hbm.at[wid], idx_v)

        def gather_cp(g, b):
            return pltpu.make_async_copy(
                table_hbm.at[idx_v.at[g]], rows_v.at[b], gsem.at[b]
            )

        for b in range(NBUF):
            gather_cp(b, b).start()

        def body(j, carry):
            for b in range(NBUF):
                g = j * NBUF + b
                gather_cp(g, b).wait()
                pltpu.sync_copy(
                    rows_v.at[b], out_hbm.at[pl.ds(base + g * CHUNK, CHUNK)]
                )
                nxt = g + NBUF

                @pl.when(nxt < nchunk)
                def _start_next():
                    gather_cp(nxt, b).start()

            return carry

        lax.fori_loop(0, ngroup, body, None)

    return gather_kernel


def kernel(vocab_id_list, table):
    batch, hist = vocab_id_list.shape
    total = batch * hist
    zero_i = lax.optimization_barrier(jnp.int32(0))
    one_f = lax.optimization_barrier(jnp.float32(1.0))
    idx = (vocab_id_list.astype(jnp.int32) + zero_i).reshape(
        NUM_WORKERS, total // NUM_WORKERS // CHUNK, CHUNK
    )
    table = table * one_f
    out = _make_kernel(total)(idx, table)
    return out.reshape(batch, hist, EMBED_DIM) * one_f


# speedup vs baseline: 2.0865x; 2.0865x over previous
import functools

import jax
import jax.numpy as jnp
from jax import lax
from jax.experimental import pallas as pl
from jax.experimental.pallas import tpu as pltpu
from jax.experimental.pallas import tpu_sc as plsc

EMBED_DIM = 32
NUM_CORES = 2
NUM_SUBCORES = 16
NUM_WORKERS = NUM_CORES * NUM_SUBCORES

NBUF = 4


@functools.lru_cache(maxsize=None)
def _make_kernel(batch, hist):
    b_per_w = batch // NUM_WORKERS
    ngroup = b_per_w // NBUF
    mesh = plsc.VectorSubcoreMesh(core_axis_name="c", subcore_axis_name="s")

    @functools.partial(
        pl.kernel,
        mesh=mesh,
        out_type=jax.ShapeDtypeStruct((batch, hist, EMBED_DIM), jnp.float32),
        compiler_params=pltpu.CompilerParams(use_tc_tiling_on_sc=False),
        scratch_types=[
            pltpu.VMEM((b_per_w, hist), jnp.int32),
            pltpu.VMEM((NBUF, hist, EMBED_DIM), jnp.float32),
            pltpu.SemaphoreType.DMA((NBUF,)),
        ],
    )
    def gather_kernel(idx_hbm, table_hbm, out_hbm, idx_v, rows_v, gsem):
        wid = lax.axis_index("s") * NUM_CORES + lax.axis_index("c")
        base = wid * b_per_w
        pltpu.sync_copy(idx_hbm.at[pl.ds(base, b_per_w)], idx_v)

        def gather_cp(i, b):
            return pltpu.make_async_copy(
                table_hbm.at[idx_v.at[i]], rows_v.at[b], gsem.at[b]
            )

        for b in range(NBUF):
            gather_cp(b, b).start()

        def body(j, carry):
            for b in range(NBUF):
                i = j * NBUF + b
                gather_cp(i, b).wait()
                pltpu.sync_copy(rows_v.at[b], out_hbm.at[base + i])
                nxt = i + NBUF

                @pl.when(nxt < b_per_w)
                def _start_next():
                    gather_cp(nxt, b).start()

            return carry

        lax.fori_loop(0, ngroup, body, None)

    return gather_kernel


def kernel(vocab_id_list, table):
    batch, hist = vocab_id_list.shape
    idx = vocab_id_list.astype(jnp.int32)
    return _make_kernel(batch, hist)(idx, table)


# speedup vs baseline: 2.1808x; 1.0452x over previous
import functools

import jax
import jax.numpy as jnp
from jax import lax
from jax.experimental import pallas as pl
from jax.experimental.pallas import tpu as pltpu
from jax.experimental.pallas import tpu_sc as plsc

EMBED_DIM = 32
NUM_CORES = 2
NUM_SUBCORES = 16
NUM_WORKERS = NUM_CORES * NUM_SUBCORES

NBUF = 8


@functools.lru_cache(maxsize=None)
def _make_kernel(batch, hist):
    b_per_w = batch // NUM_WORKERS
    ngroup = b_per_w // NBUF
    mesh = plsc.VectorSubcoreMesh(core_axis_name="c", subcore_axis_name="s")

    @functools.partial(
        pl.kernel,
        mesh=mesh,
        out_type=jax.ShapeDtypeStruct((batch, hist, EMBED_DIM), jnp.float32),
        compiler_params=pltpu.CompilerParams(use_tc_tiling_on_sc=False),
        scratch_types=[
            pltpu.VMEM((b_per_w, hist), jnp.int32),
            pltpu.VMEM((NBUF, hist, EMBED_DIM), jnp.float32),
            pltpu.SemaphoreType.DMA((NBUF,)),
        ],
    )
    def gather_kernel(idx_hbm, table_hbm, out_hbm, idx_v, rows_v, gsem):
        wid = lax.axis_index("s") * NUM_CORES + lax.axis_index("c")
        base = wid * b_per_w
        pltpu.sync_copy(idx_hbm.at[pl.ds(base, b_per_w)], idx_v)

        def gather_cp(i, b):
            return pltpu.make_async_copy(
                table_hbm.at[idx_v.at[i]], rows_v.at[b], gsem.at[b]
            )

        for b in range(NBUF):
            gather_cp(b, b).start()

        def body(j, carry):
            for b in range(NBUF):
                i = j * NBUF + b
                gather_cp(i, b).wait()
                pltpu.sync_copy(rows_v.at[b], out_hbm.at[base + i])
                nxt = i + NBUF

                @pl.when(nxt < b_per_w)
                def _start_next():
                    gather_cp(nxt, b).start()

            return carry

        lax.fori_loop(0, ngroup, body, None)

    return gather_kernel


def kernel(vocab_id_list, table):
    batch, hist = vocab_id_list.shape
    idx = vocab_id_list.astype(jnp.int32)
    return _make_kernel(batch, hist)(idx, table)


# speedup vs baseline: 2.1883x; 1.0034x over previous
import functools

import jax
import jax.numpy as jnp
from jax import lax
from jax.experimental import pallas as pl
from jax.experimental.pallas import tpu as pltpu
from jax.experimental.pallas import tpu_sc as plsc

EMBED_DIM = 32
NUM_CORES = 2
NUM_SUBCORES = 16
NUM_WORKERS = NUM_CORES * NUM_SUBCORES

NBUF = 16


@functools.lru_cache(maxsize=None)
def _make_kernel(batch, hist):
    b_per_w = batch // NUM_WORKERS
    ngroup = b_per_w // NBUF
    mesh = plsc.VectorSubcoreMesh(core_axis_name="c", subcore_axis_name="s")

    @functools.partial(
        pl.kernel,
        mesh=mesh,
        out_type=jax.ShapeDtypeStruct((batch, hist, EMBED_DIM), jnp.float32),
        compiler_params=pltpu.CompilerParams(use_tc_tiling_on_sc=False),
        scratch_types=[
            pltpu.VMEM((b_per_w, hist), jnp.int32),
            pltpu.VMEM((NBUF, hist, EMBED_DIM), jnp.float32),
            pltpu.SemaphoreType.DMA((NBUF,)),
        ],
    )
    def gather_kernel(idx_hbm, table_hbm, out_hbm, idx_v, rows_v, gsem):
        wid = lax.axis_index("s") * NUM_CORES + lax.axis_index("c")
        base = wid * b_per_w
        pltpu.sync_copy(idx_hbm.at[pl.ds(base, b_per_w)], idx_v)

        def gather_cp(i, b):
            return pltpu.make_async_copy(
                table_hbm.at[idx_v.at[i]], rows_v.at[b], gsem.at[b]
            )

        for b in range(NBUF):
            gather_cp(b, b).start()

        def body(j, carry):
            for b in range(NBUF):
                i = j * NBUF + b
                gather_cp(i, b).wait()
                pltpu.sync_copy(rows_v.at[b], out_hbm.at[base + i])
                nxt = i + NBUF

                @pl.when(nxt < b_per_w)
                def _start_next():
                    gather_cp(nxt, b).start()

            return carry

        lax.fori_loop(0, ngroup, body, None)

    return gather_kernel


def kernel(vocab_id_list, table):
    batch, hist = vocab_id_list.shape
    idx = vocab_id_list.astype(jnp.int32)
    return _make_kernel(batch, hist)(idx, table)
